# baseline (speedup 1.0000x reference)
import functools

import jax
import jax.numpy as jnp
from jax import lax
from jax.experimental import pallas as pl
from jax.experimental.pallas import tpu as pltpu
from jax.experimental.pallas import tpu_sc as plsc

N = 10000
E = 320000
D = 128
NPAD = 10240
NC, NS = 2, 16
NW = NC * NS
PADROWS_PER_SUB = NPAD // NS

CH = 64
NBUF = 3
EW_PER_W = 10240
EPADTOT = NW * EW_PER_W
EBLK = 2048
CPB = EBLK // CH
NEBLK = EW_PER_W // EBLK

ED_PER_S = EPADTOT // NS
NDBLK = ED_PER_S // EBLK


def _zero_vec16(ref, nwords):
    z = jnp.zeros((16,), jnp.float32)

    def body(i, _):
        ref[pl.ds(i * 16, 16)] = z
        return 0

    lax.fori_loop(0, nwords // 16, body, 0)


def _zero_rows(ref, nrows):
    z = jnp.zeros((16,), jnp.float32)

    def body(r, _):
        for dblk in range(8):
            ref[r, pl.ds(dblk * 16, 16)] = z
        return 0

    lax.fori_loop(0, nrows, body, 0)


def _rsqrt16(d):
    m = d
    y = jnp.full((16,), 1.0, jnp.float32)
    for _ in range(10):
        c = m > 4.0
        m = jnp.where(c, m * 0.25, m)
        y = jnp.where(c, y * 0.5, y)
    r = jnp.full((16,), 0.7, jnp.float32)
    for _ in range(5):
        r = r * (1.5 - 0.5 * m * r * r)
    return y * r


def _copy_to_2d(src1d, dst2d, nrows, width):
    def body(r, _):
        for c in range(width // 16):
            dst2d[r, pl.ds(c * 16, 16)] = src1d[pl.ds(r * width + c * 16, 16)]
        return 0

    lax.fori_loop(0, nrows, body, 0)


def _scale_chunk(xb, wbuf, rbuf, ebase, use_dis, dis_vmem):
    def group_body(g, _):
        ev = wbuf[pl.ds(ebase + g * 16, 16)]
        if use_dis:
            rv = rbuf[pl.ds(ebase + g * 16, 16)]
            ev = ev * plsc.load_gather(dis_vmem, [rv])
        for j in range(16):
            er = g * 16 + j
            s = ev[j]
            for dblk in range(8):
                xb[er, pl.ds(dblk * 16, 16)] = xb[er, pl.ds(dblk * 16, 16)] * s
        return 0

    lax.fori_loop(0, CH // 16, group_body, 0)


def _agg_block(tab_hbm, sh_agg, rbuf, wbuf, cix2d, xbs, gsems, ssems,
               use_dis, dis_vmem):
    def gidx(i):
        return rbuf.at[pl.ds(i * CH, CH)]

    for b in range(2):
        pltpu.async_copy(tab_hbm.at[gidx(b)], xbs[b], gsems[b])

    def step(si, _):
        for k in range(NBUF):
            i = si * NBUF + k

            @pl.when(i < CPB)
            def _():
                b = k
                pltpu.make_async_copy(
                    tab_hbm.at[gidx(i)], xbs[b], gsems[b]).wait()
                _scale_chunk(xbs[b], wbuf, rbuf, i * CH, use_dis, dis_vmem)
                pltpu.async_copy(xbs[b], sh_agg.at[cix2d.at[i]], ssems[b],
                                 add=True)

                @pl.when(i + 2 < CPB)
                def _():
                    b2 = (k + 2) % NBUF

                    @pl.when(i >= 1)
                    def _():
                        pltpu.make_async_copy(
                            xbs[b2], sh_agg.at[cix2d.at[i - 1]],
                            ssems[b2]).wait()

                    pltpu.async_copy(
                        tab_hbm.at[gidx(i + 2)], xbs[b2], gsems[b2])
        return 0

    nsteps = (CPB + NBUF - 1) // NBUF
    lax.fori_loop(0, nsteps, step, 0)
    for i in range(CPB - 3, CPB):
        pltpu.make_async_copy(
            xbs[i % NBUF], sh_agg.at[cix2d.at[i]], ssems[i % NBUF]).wait()


def _load_block(row_hbm, col_hbm, ew_hbm, rbuf, wbuf, cix2d, off):
    pltpu.sync_copy(col_hbm.at[pl.ds(off, EBLK)], rbuf)
    _copy_to_2d(rbuf, cix2d, CPB, CH)
    pltpu.sync_copy(row_hbm.at[pl.ds(off, EBLK)], rbuf)
    pltpu.sync_copy(ew_hbm.at[pl.ds(off, EBLK)], wbuf)


def _agg_all_blocks(tab_hbm, sh_agg, row_hbm, col_hbm, ew_hbm, ebase0,
                    rbuf, wbuf, cix2d, xbs, gsems, ssems, use_dis, dis_vmem):
    def blk_body(blk, _):
        _load_block(row_hbm, col_hbm, ew_hbm, rbuf, wbuf, cix2d,
                    ebase0 + blk * EBLK)
        _agg_block(tab_hbm, sh_agg, rbuf, wbuf, cix2d, xbs, gsems, ssems,
                   use_dis, dis_vmem)
        return 0

    lax.fori_loop(0, NEBLK, blk_body, 0)


def _zero_accum(sh_agg, sh_deg, xb0, wbuf, sid, zero_deg):
    _zero_rows(xb0, CH)
    for b in range(PADROWS_PER_SUB // CH):
        pltpu.sync_copy(
            xb0, sh_agg.at[pl.ds(sid * PADROWS_PER_SUB + b * CH, CH), :])
    if zero_deg:
        _zero_vec16(wbuf, PADROWS_PER_SUB)
        pltpu.sync_copy(wbuf.at[pl.ds(0, PADROWS_PER_SUB)],
                        sh_deg.at[pl.ds(sid * PADROWS_PER_SUB,
                                        PADROWS_PER_SUB)])


def _writeout_partial(sh_agg, out_hbm, cid, sid):
    base = sid * PADROWS_PER_SUB
    pltpu.sync_copy(sh_agg.at[pl.ds(base, PADROWS_PER_SUB), :],
                    out_hbm.at[cid, pl.ds(base, PADROWS_PER_SUB), :])


def _sc_layer1(row, col, ew, x):
    mesh = plsc.VectorSubcoreMesh(core_axis_name="c", subcore_axis_name="s")

    @functools.partial(
        pl.kernel,
        out_type=[jax.ShapeDtypeStruct((NC, NPAD, D), jnp.float32),
                  jax.ShapeDtypeStruct((NPAD,), jnp.float32)],
        mesh=mesh,
        compiler_params=pltpu.CompilerParams(needs_layout_passes=False),
        scratch_types=[
            pltpu.VMEM_SHARED((NPAD, D), jnp.float32),
            pltpu.VMEM_SHARED((NPAD,), jnp.float32),
            pltpu.VMEM((NPAD,), jnp.float32),
            pltpu.VMEM((EBLK,), jnp.int32),
            pltpu.VMEM((EBLK,), jnp.float32),
            pltpu.VMEM((CPB, CH), jnp.int32),
            pltpu.VMEM((CH, D), jnp.float32),
            pltpu.VMEM((CH, D), jnp.float32),
            pltpu.VMEM((CH, D), jnp.float32),
            pltpu.SemaphoreType.DMA,
            pltpu.SemaphoreType.DMA,
            pltpu.SemaphoreType.DMA,
            pltpu.SemaphoreType.DMA,
            pltpu.SemaphoreType.DMA,
            pltpu.SemaphoreType.DMA,
            pltpu.SemaphoreType.DMA,
        ],
    )
    def kern(row_hbm, col_hbm, ew_hbm, x_hbm, p_hbm, dis_hbm,
             sh_agg, sh_deg, dis_vmem, rbuf, wbuf, cix2d, xb0, xb1, xb2,
             g0, g1, g2, s0, s1, s2, dsem):
        cid = lax.axis_index("c")
        sid = lax.axis_index("s")
        wid = cid * NS + sid
        xbs = [xb0, xb1, xb2]
        gsems = [g0, g1, g2]
        ssems = [s0, s1, s2]

        _zero_accum(sh_agg, sh_deg, xb0, wbuf, sid, True)
        plsc.subcore_barrier()

        def deg_block(blk, _):
            off = sid * ED_PER_S + blk * EBLK
            pltpu.sync_copy(col_hbm.at[pl.ds(off, EBLK)], rbuf)
            _copy_to_2d(rbuf, cix2d, CPB, CH)
            pltpu.sync_copy(ew_hbm.at[pl.ds(off, EBLK)], wbuf)
            for k in range(CPB):
                pltpu.async_copy(wbuf.at[pl.ds(k * CH, CH)],
                                 sh_deg.at[cix2d.at[k]], dsem, add=True)
            for k in range(CPB):
                pltpu.make_async_copy(wbuf.at[pl.ds(k * CH, CH)],
                                      sh_deg.at[cix2d.at[k]], dsem).wait()
            return 0

        lax.fori_loop(0, NDBLK, deg_block, 0)
        plsc.subcore_barrier()

        nbase = sid * PADROWS_PER_SUB
        pltpu.sync_copy(sh_deg.at[pl.ds(nbase, PADROWS_PER_SUB)],
                        wbuf.at[pl.ds(0, PADROWS_PER_SUB)])

        def dis_body(i, _):
            dv = wbuf[pl.ds(i * 16, 16)] + 1.0
            wbuf[pl.ds(i * 16, 16)] = _rsqrt16(dv)
            return 0

        lax.fori_loop(0, PADROWS_PER_SUB // 16, dis_body, 0)
        pltpu.sync_copy(wbuf.at[pl.ds(0, PADROWS_PER_SUB)],
                        sh_deg.at[pl.ds(nbase, PADROWS_PER_SUB)])

        @pl.when(cid == 0)
        def _():
            pltpu.sync_copy(wbuf.at[pl.ds(0, PADROWS_PER_SUB)],
                            dis_hbm.at[pl.ds(nbase, PADROWS_PER_SUB)])

        plsc.subcore_barrier()

        pltpu.sync_copy(sh_deg, dis_vmem)
        _agg_all_blocks(x_hbm, sh_agg, row_hbm, col_hbm, ew_hbm,
                        wid * EW_PER_W, rbuf, wbuf, cix2d, xbs, gsems,
                        ssems, True, dis_vmem)
        plsc.subcore_barrier()

        _writeout_partial(sh_agg, p_hbm, cid, sid)

    return kern(row, col, ew, x)


def _sc_layer2(row, col, ew, g):
    mesh = plsc.VectorSubcoreMesh(core_axis_name="c", subcore_axis_name="s")

    @functools.partial(
        pl.kernel,
        out_type=[jax.ShapeDtypeStruct((NC, NPAD, D), jnp.float32)],
        mesh=mesh,
        compiler_params=pltpu.CompilerParams(needs_layout_passes=False),
        scratch_types=[
            pltpu.VMEM_SHARED((NPAD, D), jnp.float32),
            pltpu.VMEM((EBLK,), jnp.int32),
            pltpu.VMEM((EBLK,), jnp.float32),
            pltpu.VMEM((CPB, CH), jnp.int32),
            pltpu.VMEM((CH, D), jnp.float32),
            pltpu.VMEM((CH, D), jnp.float32),
            pltpu.VMEM((CH, D), jnp.float32),
            pltpu.SemaphoreType.DMA,
            pltpu.SemaphoreType.DMA,
            pltpu.SemaphoreType.DMA,
            pltpu.SemaphoreType.DMA,
            pltpu.SemaphoreType.DMA,
            pltpu.SemaphoreType.DMA,
        ],
    )
    def kern(row_hbm, col_hbm, ew_hbm, g_hbm, q_hbm,
             sh_agg, rbuf, wbuf, cix2d, xb0, xb1, xb2,
             g0, g1, g2, s0, s1, s2):
        cid = lax.axis_index("c")
        sid = lax.axis_index("s")
        wid = cid * NS + sid
        xbs = [xb0, xb1, xb2]
        gsems = [g0, g1, g2]
        ssems = [s0, s1, s2]

        _zero_accum(sh_agg, None, xb0, wbuf, sid, False)
        plsc.subcore_barrier()

        _agg_all_blocks(g_hbm, sh_agg, row_hbm, col_hbm, ew_hbm,
                        wid * EW_PER_W, rbuf, wbuf, cix2d, xbs, gsems,
                        ssems, False, None)
        plsc.subcore_barrier()

        _writeout_partial(sh_agg, q_hbm, cid, sid)

    return kern(row, col, ew, g)[0]


BLK = 1000


def _tc_mid_body(x, p0, p1, dis, W1, Wr1, W2, Wr2, b1, br1, b2, br2,
                 g_o, base_o):
    xv = x[...]
    disv = dis[...]
    a = disv * (p0[...] + p1[...]) + (disv * disv) * xv
    h = jnp.maximum(
        jnp.dot(a, W1[...], preferred_element_type=jnp.float32)
        + jnp.dot(xv, Wr1[...], preferred_element_type=jnp.float32)
        + b1[...] + br1[...], 0.0)
    g = disv * jnp.dot(h, W2[...], preferred_element_type=jnp.float32)
    base_o[...] = (jnp.dot(h, Wr2[...], preferred_element_type=jnp.float32)
                   + br2[...] + b2[...] + disv * g)
    g_o[...] = g


def _tc_mid(x, p0, p1, dis, W1, Wr1, W2, Wr2, b1, br1, b2, br2):
    nblk = N // BLK
    rows = lambda i: (i, 0)
    whole = lambda i: (0, 0)
    return pl.pallas_call(
        _tc_mid_body,
        grid=(nblk,),
        in_specs=[
            pl.BlockSpec((BLK, D), rows),
            pl.BlockSpec((BLK, D), rows),
            pl.BlockSpec((BLK, D), rows),
            pl.BlockSpec((BLK, 1), rows),
            pl.BlockSpec((D, 1024), whole),
            pl.BlockSpec((D, 1024), whole),
            pl.BlockSpec((1024, D), whole),
            pl.BlockSpec((1024, D), whole),
            pl.BlockSpec((1, 1024), whole),
            pl.BlockSpec((1, 1024), whole),
            pl.BlockSpec((1, D), whole),
            pl.BlockSpec((1, D), whole),
        ],
        out_specs=[pl.BlockSpec((BLK, D), rows),
                   pl.BlockSpec((BLK, D), rows)],
        out_shape=[jax.ShapeDtypeStruct((N, D), jnp.float32),
                   jax.ShapeDtypeStruct((N, D), jnp.float32)],
    )(x, p0, p1, dis, W1, Wr1, W2, Wr2, b1, br1, b2, br2)


def _tc_final_body(base, q0, q1, dis, out_o):
    out_o[...] = base[...] + dis[...] * (q0[...] + q1[...])


def _tc_final(base, q0, q1, dis):
    nblk = N // BLK
    rows = lambda i: (i, 0)
    return pl.pallas_call(
        _tc_final_body,
        grid=(nblk,),
        in_specs=[pl.BlockSpec((BLK, D), rows),
                  pl.BlockSpec((BLK, D), rows),
                  pl.BlockSpec((BLK, D), rows),
                  pl.BlockSpec((BLK, 1), rows)],
        out_specs=pl.BlockSpec((BLK, D), rows),
        out_shape=jax.ShapeDtypeStruct((N, D), jnp.float32),
    )(base, q0, q1, dis)


def kernel(x, edge_index, edge_attr, W1, b1, W2, b2, Wr1, br1, Wr2, br2):
    npad_e = EPADTOT - E
    row = jnp.concatenate(
        [edge_index[0], jnp.zeros((npad_e,), jnp.int32)])
    col = jnp.concatenate(
        [edge_index[1], jnp.full((npad_e,), NPAD - 1, jnp.int32)])
    ew = jnp.concatenate(
        [edge_attr, jnp.zeros((npad_e,), jnp.float32)])

    p, dis_pad = _sc_layer1(row, col, ew, x)
    dis = dis_pad[:N].reshape(N, 1)

    g, base = _tc_mid(x, p[0, :N], p[1, :N], dis,
                      W1, Wr1, W2, Wr2,
                      b1.reshape(1, -1), br1.reshape(1, -1),
                      b2.reshape(1, -1), br2.reshape(1, -1))

    q = _sc_layer2(row, col, ew, g)

    return _tc_final(base, q[0, :N], q[1, :N], dis)


# speedup vs baseline: 1.0046x; 1.0046x over previous
import functools

import jax
import jax.numpy as jnp
from jax import lax
from jax.experimental import pallas as pl
from jax.experimental.pallas import tpu as pltpu
from jax.experimental.pallas import tpu_sc as plsc

N = 10000
E = 320000
D = 128
NPAD = 10240
NC, NS = 2, 16
NW = NC * NS
PADROWS_PER_SUB = NPAD // NS

CH = 64
NBUF = 3
EW_PER_W = 10240
EPADTOT = NW * EW_PER_W
EBLK = 2048
CPB = EBLK // CH
NEBLK = EW_PER_W // EBLK

ED_PER_S = EPADTOT // NS
NDBLK = ED_PER_S // EBLK


def _zero_vec16(ref, nwords):
    z = jnp.zeros((16,), jnp.float32)

    def body(i, _):
        ref[pl.ds(i * 16, 16)] = z
        return 0

    lax.fori_loop(0, nwords // 16, body, 0)


def _zero_rows(ref, nrows):
    z = jnp.zeros((16,), jnp.float32)

    def body(r, _):
        for dblk in range(8):
            ref[r, pl.ds(dblk * 16, 16)] = z
        return 0

    lax.fori_loop(0, nrows, body, 0)


def _rsqrt16(d):
    m = d
    y = jnp.full((16,), 1.0, jnp.float32)
    for _ in range(10):
        c = m > 4.0
        m = jnp.where(c, m * 0.25, m)
        y = jnp.where(c, y * 0.5, y)
    r = jnp.full((16,), 0.7, jnp.float32)
    for _ in range(5):
        r = r * (1.5 - 0.5 * m * r * r)
    return y * r


def _copy_to_2d(src1d, dst2d, nrows, width):
    def body(r, _):
        for c in range(width // 16):
            dst2d[r, pl.ds(c * 16, 16)] = src1d[pl.ds(r * width + c * 16, 16)]
        return 0

    lax.fori_loop(0, nrows, body, 0)


def _scale_chunk(xb, wbuf, rbuf, ebase, use_dis, dis_vmem):
    def group_body(g, _):
        ev = wbuf[pl.ds(ebase + g * 16, 16)]
        if use_dis:
            rv = rbuf[pl.ds(ebase + g * 16, 16)]
            ev = ev * plsc.load_gather(dis_vmem, [rv])
        for j in range(16):
            er = g * 16 + j
            s = ev[j]
            for dblk in range(8):
                xb[er, pl.ds(dblk * 16, 16)] = xb[er, pl.ds(dblk * 16, 16)] * s
        return 0

    lax.fori_loop(0, CH // 16, group_body, 0)


def _agg_block(tab_hbm, sh_agg, rbuf, wbuf, cix2d, xbs, gsems, ssems,
               use_dis, dis_vmem):
    def gidx(i):
        return rbuf.at[pl.ds(i * CH, CH)]

    for b in range(2):
        pltpu.async_copy(tab_hbm.at[gidx(b)], xbs[b], gsems[b])

    def step(si, _):
        for k in range(NBUF):
            i = si * NBUF + k

            @pl.when(i < CPB)
            def _():
                b = k
                pltpu.make_async_copy(
                    tab_hbm.at[gidx(i)], xbs[b], gsems[b]).wait()
                _scale_chunk(xbs[b], wbuf, rbuf, i * CH, use_dis, dis_vmem)
                pltpu.async_copy(xbs[b], sh_agg.at[cix2d.at[i]], ssems[b],
                                 add=True)

                @pl.when(i + 2 < CPB)
                def _():
                    b2 = (k + 2) % NBUF

                    @pl.when(i >= 1)
                    def _():
                        pltpu.make_async_copy(
                            xbs[b2], sh_agg.at[cix2d.at[i - 1]],
                            ssems[b2]).wait()

                    pltpu.async_copy(
                        tab_hbm.at[gidx(i + 2)], xbs[b2], gsems[b2])
        return 0

    nsteps = (CPB + NBUF - 1) // NBUF
    lax.fori_loop(0, nsteps, step, 0)
    for i in range(CPB - 3, CPB):
        pltpu.make_async_copy(
            xbs[i % NBUF], sh_agg.at[cix2d.at[i]], ssems[i % NBUF]).wait()


def _load_block(row_hbm, col_hbm, ew_hbm, rbuf, wbuf, cix2d, off):
    pltpu.sync_copy(col_hbm.at[pl.ds(off, EBLK)], rbuf)
    _copy_to_2d(rbuf, cix2d, CPB, CH)
    pltpu.sync_copy(row_hbm.at[pl.ds(off, EBLK)], rbuf)
    pltpu.sync_copy(ew_hbm.at[pl.ds(off, EBLK)], wbuf)


def _agg_all_blocks(tab_hbm, sh_agg, row_hbm, col_hbm, ew_hbm, ebase0,
                    rbuf, wbuf, cix2d, xbs, gsems, ssems, use_dis, dis_vmem):
    def blk_body(blk, _):
        _load_block(row_hbm, col_hbm, ew_hbm, rbuf, wbuf, cix2d,
                    ebase0 + blk * EBLK)
        _agg_block(tab_hbm, sh_agg, rbuf, wbuf, cix2d, xbs, gsems, ssems,
                   use_dis, dis_vmem)
        return 0

    lax.fori_loop(0, NEBLK, blk_body, 0)


def _zero_accum(sh_agg, sh_deg, xb0, wbuf, sid, zero_deg):
    _zero_rows(xb0, CH)
    for b in range(PADROWS_PER_SUB // CH):
        pltpu.sync_copy(
            xb0, sh_agg.at[pl.ds(sid * PADROWS_PER_SUB + b * CH, CH), :])
    if zero_deg:
        _zero_vec16(wbuf, PADROWS_PER_SUB)
        pltpu.sync_copy(wbuf.at[pl.ds(0, PADROWS_PER_SUB)],
                        sh_deg.at[pl.ds(sid * PADROWS_PER_SUB,
                                        PADROWS_PER_SUB)])


def _writeout_partial(sh_agg, out_hbm, cid, sid):
    base = sid * PADROWS_PER_SUB
    pltpu.sync_copy(sh_agg.at[pl.ds(base, PADROWS_PER_SUB), :],
                    out_hbm.at[cid, pl.ds(base, PADROWS_PER_SUB), :])


def _sc_layer1(row, col, ew, x):
    mesh = plsc.VectorSubcoreMesh(core_axis_name="c", subcore_axis_name="s")

    @functools.partial(
        pl.kernel,
        out_type=[jax.ShapeDtypeStruct((NC, NPAD, D), jnp.float32),
                  jax.ShapeDtypeStruct((NPAD,), jnp.float32)],
        mesh=mesh,
        compiler_params=pltpu.CompilerParams(needs_layout_passes=False),
        scratch_types=[
            pltpu.VMEM_SHARED((NPAD, D), jnp.float32),
            pltpu.VMEM_SHARED((NPAD,), jnp.float32),
            pltpu.VMEM((NPAD,), jnp.float32),
            pltpu.VMEM((EBLK,), jnp.int32),
            pltpu.VMEM((EBLK,), jnp.float32),
            pltpu.VMEM((CPB, CH), jnp.int32),
            pltpu.VMEM((CH, D), jnp.float32),
            pltpu.VMEM((CH, D), jnp.float32),
            pltpu.VMEM((CH, D), jnp.float32),
            pltpu.SemaphoreType.DMA,
            pltpu.SemaphoreType.DMA,
            pltpu.SemaphoreType.DMA,
            pltpu.SemaphoreType.DMA,
            pltpu.SemaphoreType.DMA,
            pltpu.SemaphoreType.DMA,
            pltpu.SemaphoreType.DMA,
        ],
    )
    def kern(row_hbm, col_hbm, ew_hbm, x_hbm, p_hbm, dis_hbm,
             sh_agg, sh_deg, dis_vmem, rbuf, wbuf, cix2d, xb0, xb1, xb2,
             g0, g1, g2, s0, s1, s2, dsem):
        cid = lax.axis_index("c")
        sid = lax.axis_index("s")
        wid = cid * NS + sid
        xbs = [xb0, xb1, xb2]
        gsems = [g0, g1, g2]
        ssems = [s0, s1, s2]

        _zero_accum(sh_agg, sh_deg, xb0, wbuf, sid, True)
        plsc.subcore_barrier()

        def deg_block(blk, _):
            off = sid * ED_PER_S + blk * EBLK
            pltpu.sync_copy(col_hbm.at[pl.ds(off, EBLK)], rbuf)
            _copy_to_2d(rbuf, cix2d, CPB, CH)
            pltpu.sync_copy(ew_hbm.at[pl.ds(off, EBLK)], wbuf)
            for k in range(CPB):
                pltpu.async_copy(wbuf.at[pl.ds(k * CH, CH)],
                                 sh_deg.at[cix2d.at[k]], dsem, add=True)
            for k in range(CPB):
                pltpu.make_async_copy(wbuf.at[pl.ds(k * CH, CH)],
                                      sh_deg.at[cix2d.at[k]], dsem).wait()
            return 0

        lax.fori_loop(0, NDBLK, deg_block, 0)
        plsc.subcore_barrier()

        nbase = sid * PADROWS_PER_SUB
        pltpu.sync_copy(sh_deg.at[pl.ds(nbase, PADROWS_PER_SUB)],
                        wbuf.at[pl.ds(0, PADROWS_PER_SUB)])

        def dis_body(i, _):
            dv = wbuf[pl.ds(i * 16, 16)] + 1.0
            wbuf[pl.ds(i * 16, 16)] = _rsqrt16(dv)
            return 0

        lax.fori_loop(0, PADROWS_PER_SUB // 16, dis_body, 0)
        pltpu.sync_copy(wbuf.at[pl.ds(0, PADROWS_PER_SUB)],
                        sh_deg.at[pl.ds(nbase, PADROWS_PER_SUB)])

        @pl.when(cid == 0)
        def _():
            pltpu.sync_copy(wbuf.at[pl.ds(0, PADROWS_PER_SUB)],
                            dis_hbm.at[pl.ds(nbase, PADROWS_PER_SUB)])

        plsc.subcore_barrier()

        pltpu.sync_copy(sh_deg, dis_vmem)
        _agg_all_blocks(x_hbm, sh_agg, row_hbm, col_hbm, ew_hbm,
                        wid * EW_PER_W, rbuf, wbuf, cix2d, xbs, gsems,
                        ssems, True, dis_vmem)
        plsc.subcore_barrier()

        _writeout_partial(sh_agg, p_hbm, cid, sid)

    return kern(row, col, ew, x)


def _sc_layer2(row, col, ew, g):
    mesh = plsc.VectorSubcoreMesh(core_axis_name="c", subcore_axis_name="s")

    @functools.partial(
        pl.kernel,
        out_type=[jax.ShapeDtypeStruct((NC, NPAD, D), jnp.float32)],
        mesh=mesh,
        compiler_params=pltpu.CompilerParams(needs_layout_passes=False),
        scratch_types=[
            pltpu.VMEM_SHARED((NPAD, D), jnp.float32),
            pltpu.VMEM((EBLK,), jnp.int32),
            pltpu.VMEM((EBLK,), jnp.float32),
            pltpu.VMEM((CPB, CH), jnp.int32),
            pltpu.VMEM((CH, D), jnp.float32),
            pltpu.VMEM((CH, D), jnp.float32),
            pltpu.VMEM((CH, D), jnp.float32),
            pltpu.SemaphoreType.DMA,
            pltpu.SemaphoreType.DMA,
            pltpu.SemaphoreType.DMA,
            pltpu.SemaphoreType.DMA,
            pltpu.SemaphoreType.DMA,
            pltpu.SemaphoreType.DMA,
        ],
    )
    def kern(row_hbm, col_hbm, ew_hbm, g_hbm, q_hbm,
             sh_agg, rbuf, wbuf, cix2d, xb0, xb1, xb2,
             g0, g1, g2, s0, s1, s2):
        cid = lax.axis_index("c")
        sid = lax.axis_index("s")
        wid = cid * NS + sid
        xbs = [xb0, xb1, xb2]
        gsems = [g0, g1, g2]
        ssems = [s0, s1, s2]

        _zero_accum(sh_agg, None, xb0, wbuf, sid, False)
        plsc.subcore_barrier()

        _agg_all_blocks(g_hbm, sh_agg, row_hbm, col_hbm, ew_hbm,
                        wid * EW_PER_W, rbuf, wbuf, cix2d, xbs, gsems,
                        ssems, False, None)
        plsc.subcore_barrier()

        _writeout_partial(sh_agg, q_hbm, cid, sid)

    return kern(row, col, ew, g)[0]


BLK = 1000


def _tc_mid_body(x, p0, p1, dis, W1, Wr1, W2, Wr2, b1, br1, b2, br2,
                 g_o, base_o):
    xv = x[...]
    disv = dis[...]
    a = disv * (p0[...] + p1[...]) + (disv * disv) * xv
    h = jnp.maximum(
        jnp.dot(a, W1[...], preferred_element_type=jnp.float32)
        + jnp.dot(xv, Wr1[...], preferred_element_type=jnp.float32)
        + b1[...] + br1[...], 0.0)
    g = disv * jnp.dot(h, W2[...], preferred_element_type=jnp.float32)
    base_o[...] = (jnp.dot(h, Wr2[...], preferred_element_type=jnp.float32)
                   + br2[...] + b2[...] + disv * g)
    g_o[...] = g


def _tc_mid(x, p0, p1, dis, W1, Wr1, W2, Wr2, b1, br1, b2, br2):
    nblk = N // BLK
    rows = lambda i: (i, 0)
    whole = lambda i: (0, 0)
    return pl.pallas_call(
        _tc_mid_body,
        grid=(nblk,),
        in_specs=[
            pl.BlockSpec((BLK, D), rows),
            pl.BlockSpec((BLK, D), rows),
            pl.BlockSpec((BLK, D), rows),
            pl.BlockSpec((BLK, 1), rows),
            pl.BlockSpec((D, 1024), whole),
            pl.BlockSpec((D, 1024), whole),
            pl.BlockSpec((1024, D), whole),
            pl.BlockSpec((1024, D), whole),
            pl.BlockSpec((1, 1024), whole),
            pl.BlockSpec((1, 1024), whole),
            pl.BlockSpec((1, D), whole),
            pl.BlockSpec((1, D), whole),
        ],
        out_specs=[pl.BlockSpec((BLK, D), rows),
                   pl.BlockSpec((BLK, D), rows)],
        out_shape=[jax.ShapeDtypeStruct((N, D), jnp.float32),
                   jax.ShapeDtypeStruct((N, D), jnp.float32)],
    )(x, p0, p1, dis, W1, Wr1, W2, Wr2, b1, br1, b2, br2)


def _tc_final_body(base, q0, q1, dis, out_o):
    out_o[...] = base[...] + dis[...] * (q0[...] + q1[...])


def _tc_final(base, q0, q1, dis):
    nblk = N // BLK
    rows = lambda i: (i, 0)
    return pl.pallas_call(
        _tc_final_body,
        grid=(nblk,),
        in_specs=[pl.BlockSpec((BLK, D), rows),
                  pl.BlockSpec((BLK, D), rows),
                  pl.BlockSpec((BLK, D), rows),
                  pl.BlockSpec((BLK, 1), rows)],
        out_specs=pl.BlockSpec((BLK, D), rows),
        out_shape=jax.ShapeDtypeStruct((N, D), jnp.float32),
    )(base, q0, q1, dis)


def kernel(x, edge_index, edge_attr, W1, b1, W2, b2, Wr1, br1, Wr2, br2):
    npad_e = EPADTOT - E
    row = jnp.concatenate(
        [edge_index[0], jnp.zeros((npad_e,), jnp.int32)])
    col = jnp.concatenate(
        [edge_index[1],
         N + (jnp.arange(npad_e, dtype=jnp.int32) % (NPAD - N))])
    ew = jnp.concatenate(
        [edge_attr, jnp.zeros((npad_e,), jnp.float32)])

    p, dis_pad = _sc_layer1(row, col, ew, x)
    dis = dis_pad[:N].reshape(N, 1)

    g, base = _tc_mid(x, p[0, :N], p[1, :N], dis,
                      W1, Wr1, W2, Wr2,
                      b1.reshape(1, -1), br1.reshape(1, -1),
                      b2.reshape(1, -1), br2.reshape(1, -1))

    q = _sc_layer2(row, col, ew, g)

    return _tc_final(base, q[0, :N], q[1, :N], dis)


# speedup vs baseline: 2.4604x; 2.4493x over previous
import functools

import jax
import jax.numpy as jnp
from jax import lax
from jax.experimental import pallas as pl
from jax.experimental.pallas import tpu as pltpu
from jax.experimental.pallas import tpu_sc as plsc

N = 10000
E = 320000
D = 128
NPAD = 10240
NC, NS = 2, 16
NW = NC * NS
PADROWS_PER_SUB = NPAD // NS

CH = 64
NBUF = 3
EW_PER_W = 10240
EPADTOT = NW * EW_PER_W
EBLK = 2048
CPB = EBLK // CH
NEBLK = EW_PER_W // EBLK

ED_PER_S = EPADTOT // NS
NDBLK = ED_PER_S // EBLK


def _zero_vec16(ref, nwords):
    z = jnp.zeros((16,), jnp.float32)

    def body(i, _):
        ref[pl.ds(i * 16, 16)] = z
        return 0

    lax.fori_loop(0, nwords // 16, body, 0)


def _zero_rows(ref, nrows):
    z = jnp.zeros((16,), jnp.float32)

    def body(r, _):
        for dblk in range(8):
            ref[r, pl.ds(dblk * 16, 16)] = z
        return 0

    lax.fori_loop(0, nrows, body, 0)


def _rsqrt16(d):
    m = d
    y = jnp.full((16,), 1.0, jnp.float32)
    for _ in range(10):
        c = m > 4.0
        m = jnp.where(c, m * 0.25, m)
        y = jnp.where(c, y * 0.5, y)
    r = jnp.full((16,), 0.7, jnp.float32)
    for _ in range(5):
        r = r * (1.5 - 0.5 * m * r * r)
    return y * r


def _copy_to_2d(src1d, dst2d, nrows, width):
    def body(r, _):
        for c in range(width // 16):
            dst2d[r, pl.ds(c * 16, 16)] = src1d[pl.ds(r * width + c * 16, 16)]
        return 0

    lax.fori_loop(0, nrows, body, 0)


def _scale_chunk(xb, wbuf, rbuf, ebase, use_dis, dis_vmem):
    def group_body(g, _):
        ev = wbuf[pl.ds(ebase + g * 16, 16)]
        if use_dis:
            rv = rbuf[pl.ds(ebase + g * 16, 16)]
            ev = ev * plsc.load_gather(dis_vmem, [rv])
        for j in range(16):
            er = g * 16 + j
            s = ev[j]
            for dblk in range(8):
                xb[er, pl.ds(dblk * 16, 16)] = xb[er, pl.ds(dblk * 16, 16)] * s
        return 0

    lax.fori_loop(0, CH // 16, group_body, 0)


def _agg_block(tab_hbm, sh_agg, rbuf, wbuf, cix2d, xbs, gsems, ssems,
               use_dis, dis_vmem):
    def gidx(i):
        return rbuf.at[pl.ds(i * CH, CH)]

    for b in range(2):
        pltpu.async_copy(tab_hbm.at[gidx(b)], xbs[b], gsems[b])

    def step(si, _):
        for k in range(NBUF):
            i = si * NBUF + k

            @pl.when(i < CPB)
            def _():
                b = k
                pltpu.make_async_copy(
                    tab_hbm.at[gidx(i)], xbs[b], gsems[b]).wait()
                _scale_chunk(xbs[b], wbuf, rbuf, i * CH, use_dis, dis_vmem)
                pltpu.async_copy(xbs[b], sh_agg.at[cix2d.at[i]], ssems[b],
                                 add=True)

                @pl.when(i + 2 < CPB)
                def _():
                    b2 = (k + 2) % NBUF

                    @pl.when(i >= 1)
                    def _():
                        pltpu.make_async_copy(
                            xbs[b2], sh_agg.at[cix2d.at[i - 1]],
                            ssems[b2]).wait()

                    pltpu.async_copy(
                        tab_hbm.at[gidx(i + 2)], xbs[b2], gsems[b2])
        return 0

    nsteps = (CPB + NBUF - 1) // NBUF
    lax.fori_loop(0, nsteps, step, 0)
    for i in range(CPB - 3, CPB):
        pltpu.make_async_copy(
            xbs[i % NBUF], sh_agg.at[cix2d.at[i]], ssems[i % NBUF]).wait()


def _load_block(row_hbm, col_hbm, ew_hbm, rbuf, wbuf, cix2d, off):
    pltpu.sync_copy(col_hbm.at[pl.ds(off, EBLK)], rbuf)
    _copy_to_2d(rbuf, cix2d, CPB, CH)
    pltpu.sync_copy(row_hbm.at[pl.ds(off, EBLK)], rbuf)
    pltpu.sync_copy(ew_hbm.at[pl.ds(off, EBLK)], wbuf)


def _agg_all_blocks(tab_hbm, sh_agg, row_hbm, col_hbm, ew_hbm, ebase0,
                    rbuf, wbuf, cix2d, xbs, gsems, ssems, use_dis, dis_vmem):
    def blk_body(blk, _):
        _load_block(row_hbm, col_hbm, ew_hbm, rbuf, wbuf, cix2d,
                    ebase0 + blk * EBLK)
        _agg_block(tab_hbm, sh_agg, rbuf, wbuf, cix2d, xbs, gsems, ssems,
                   use_dis, dis_vmem)
        return 0

    lax.fori_loop(0, NEBLK, blk_body, 0)


def _zero_accum(sh_agg, sh_deg, xb0, wbuf, sid, zero_deg):
    _zero_rows(xb0, CH)
    for b in range(PADROWS_PER_SUB // CH):
        pltpu.sync_copy(
            xb0, sh_agg.at[pl.ds(sid * PADROWS_PER_SUB + b * CH, CH), :])
    if zero_deg:
        _zero_vec16(wbuf, PADROWS_PER_SUB)
        pltpu.sync_copy(wbuf.at[pl.ds(0, PADROWS_PER_SUB)],
                        sh_deg.at[pl.ds(sid * PADROWS_PER_SUB,
                                        PADROWS_PER_SUB)])


def _writeout_partial(sh_agg, out_hbm, cid, sid):
    base = sid * PADROWS_PER_SUB
    pltpu.sync_copy(sh_agg.at[pl.ds(base, PADROWS_PER_SUB), :],
                    out_hbm.at[cid, pl.ds(base, PADROWS_PER_SUB), :])


def _sc_layer1(row, col, ew, x):
    mesh = plsc.VectorSubcoreMesh(core_axis_name="c", subcore_axis_name="s")

    @functools.partial(
        pl.kernel,
        out_type=[jax.ShapeDtypeStruct((NC, NPAD, D), jnp.float32),
                  jax.ShapeDtypeStruct((NPAD,), jnp.float32)],
        mesh=mesh,
        compiler_params=pltpu.CompilerParams(needs_layout_passes=False),
        scratch_types=[
            pltpu.VMEM_SHARED((NPAD, D), jnp.float32),
            pltpu.VMEM_SHARED((NPAD,), jnp.float32),
            pltpu.VMEM((NPAD,), jnp.float32),
            pltpu.VMEM((EBLK,), jnp.int32),
            pltpu.VMEM((EBLK,), jnp.float32),
            pltpu.VMEM((CPB, CH), jnp.int32),
            pltpu.VMEM((CH, D), jnp.float32),
            pltpu.VMEM((CH, D), jnp.float32),
            pltpu.VMEM((CH, D), jnp.float32),
            pltpu.SemaphoreType.DMA,
            pltpu.SemaphoreType.DMA,
            pltpu.SemaphoreType.DMA,
            pltpu.SemaphoreType.DMA,
            pltpu.SemaphoreType.DMA,
            pltpu.SemaphoreType.DMA,
            pltpu.SemaphoreType.DMA,
        ],
    )
    def kern(row_hbm, col_hbm, ew_hbm, x_hbm, p_hbm, dis_hbm,
             sh_agg, sh_deg, dis_vmem, rbuf, wbuf, cix2d, xb0, xb1, xb2,
             g0, g1, g2, s0, s1, s2, dsem):
        cid = lax.axis_index("c")
        sid = lax.axis_index("s")
        wid = cid * NS + sid
        xbs = [xb0, xb1, xb2]
        gsems = [g0, g1, g2]
        ssems = [s0, s1, s2]

        _zero_accum(sh_agg, sh_deg, xb0, wbuf, sid, True)
        plsc.subcore_barrier()

        def deg_block(blk, _):
            off = sid * ED_PER_S + blk * EBLK
            pltpu.sync_copy(col_hbm.at[pl.ds(off, EBLK)], rbuf)
            _copy_to_2d(rbuf, cix2d, CPB, CH)
            pltpu.sync_copy(ew_hbm.at[pl.ds(off, EBLK)], wbuf)
            for k in range(CPB):
                pltpu.async_copy(wbuf.at[pl.ds(k * CH, CH)],
                                 sh_deg.at[cix2d.at[k]], dsem, add=True)
            for k in range(CPB):
                pltpu.make_async_copy(wbuf.at[pl.ds(k * CH, CH)],
                                      sh_deg.at[cix2d.at[k]], dsem).wait()
            return 0

        lax.fori_loop(0, NDBLK, deg_block, 0)
        plsc.subcore_barrier()

        nbase = sid * PADROWS_PER_SUB
        pltpu.sync_copy(sh_deg.at[pl.ds(nbase, PADROWS_PER_SUB)],
                        wbuf.at[pl.ds(0, PADROWS_PER_SUB)])

        def dis_body(i, _):
            dv = wbuf[pl.ds(i * 16, 16)] + 1.0
            wbuf[pl.ds(i * 16, 16)] = _rsqrt16(dv)
            return 0

        lax.fori_loop(0, PADROWS_PER_SUB // 16, dis_body, 0)
        pltpu.sync_copy(wbuf.at[pl.ds(0, PADROWS_PER_SUB)],
                        sh_deg.at[pl.ds(nbase, PADROWS_PER_SUB)])

        @pl.when(cid == 0)
        def _():
            pltpu.sync_copy(wbuf.at[pl.ds(0, PADROWS_PER_SUB)],
                            dis_hbm.at[pl.ds(nbase, PADROWS_PER_SUB)])

        plsc.subcore_barrier()

        pltpu.sync_copy(sh_deg, dis_vmem)
        _agg_all_blocks(x_hbm, sh_agg, row_hbm, col_hbm, ew_hbm,
                        wid * EW_PER_W, rbuf, wbuf, cix2d, xbs, gsems,
                        ssems, True, dis_vmem)
        plsc.subcore_barrier()

        _writeout_partial(sh_agg, p_hbm, cid, sid)

    return kern(row, col, ew, x)


def _sc_layer2(row, col, ew, g):
    mesh = plsc.VectorSubcoreMesh(core_axis_name="c", subcore_axis_name="s")

    @functools.partial(
        pl.kernel,
        out_type=[jax.ShapeDtypeStruct((NC, NPAD, D), jnp.float32)],
        mesh=mesh,
        compiler_params=pltpu.CompilerParams(needs_layout_passes=False),
        scratch_types=[
            pltpu.VMEM_SHARED((NPAD, D), jnp.float32),
            pltpu.VMEM((EBLK,), jnp.int32),
            pltpu.VMEM((EBLK,), jnp.float32),
            pltpu.VMEM((CPB, CH), jnp.int32),
            pltpu.VMEM((CH, D), jnp.float32),
            pltpu.VMEM((CH, D), jnp.float32),
            pltpu.VMEM((CH, D), jnp.float32),
            pltpu.SemaphoreType.DMA,
            pltpu.SemaphoreType.DMA,
            pltpu.SemaphoreType.DMA,
            pltpu.SemaphoreType.DMA,
            pltpu.SemaphoreType.DMA,
            pltpu.SemaphoreType.DMA,
        ],
    )
    def kern(row_hbm, col_hbm, ew_hbm, g_hbm, q_hbm,
             sh_agg, rbuf, wbuf, cix2d, xb0, xb1, xb2,
             g0, g1, g2, s0, s1, s2):
        cid = lax.axis_index("c")
        sid = lax.axis_index("s")
        wid = cid * NS + sid
        xbs = [xb0, xb1, xb2]
        gsems = [g0, g1, g2]
        ssems = [s0, s1, s2]

        _zero_accum(sh_agg, None, xb0, wbuf, sid, False)
        plsc.subcore_barrier()

        _agg_all_blocks(g_hbm, sh_agg, row_hbm, col_hbm, ew_hbm,
                        wid * EW_PER_W, rbuf, wbuf, cix2d, xbs, gsems,
                        ssems, False, None)
        plsc.subcore_barrier()

        _writeout_partial(sh_agg, q_hbm, cid, sid)

    return kern(row, col, ew, g)[0]


BLK = 1000


def _tc_mid_body(x, p0, p1, dis, W1, Wr1, W2, Wr2, b1, br1, b2, br2,
                 g_o, base_o):
    xv = x[...]
    disv = dis[...]
    a = disv * (p0[...] + p1[...]) + (disv * disv) * xv
    h = jnp.maximum(
        jnp.dot(a, W1[...], preferred_element_type=jnp.float32)
        + jnp.dot(xv, Wr1[...], preferred_element_type=jnp.float32)
        + b1[...] + br1[...], 0.0)
    g = disv * jnp.dot(h, W2[...], preferred_element_type=jnp.float32)
    base_o[...] = (jnp.dot(h, Wr2[...], preferred_element_type=jnp.float32)
                   + br2[...] + b2[...] + disv * g)
    g_o[...] = g


def _tc_mid(x, p0, p1, dis, W1, Wr1, W2, Wr2, b1, br1, b2, br2):
    nblk = N // BLK
    rows = lambda i: (i, 0)
    whole = lambda i: (0, 0)
    return pl.pallas_call(
        _tc_mid_body,
        grid=(nblk,),
        in_specs=[
            pl.BlockSpec((BLK, D), rows),
            pl.BlockSpec((BLK, D), rows),
            pl.BlockSpec((BLK, D), rows),
            pl.BlockSpec((BLK, 1), rows),
            pl.BlockSpec((D, 1024), whole),
            pl.BlockSpec((D, 1024), whole),
            pl.BlockSpec((1024, D), whole),
            pl.BlockSpec((1024, D), whole),
            pl.BlockSpec((1, 1024), whole),
            pl.BlockSpec((1, 1024), whole),
            pl.BlockSpec((1, D), whole),
            pl.BlockSpec((1, D), whole),
        ],
        out_specs=[pl.BlockSpec((BLK, D), rows),
                   pl.BlockSpec((BLK, D), rows)],
        out_shape=[jax.ShapeDtypeStruct((N, D), jnp.float32),
                   jax.ShapeDtypeStruct((N, D), jnp.float32)],
    )(x, p0, p1, dis, W1, Wr1, W2, Wr2, b1, br1, b2, br2)


def _tc_final_body(base, q0, q1, dis, out_o):
    out_o[...] = base[...] + dis[...] * (q0[...] + q1[...])


def _tc_final(base, q0, q1, dis):
    nblk = N // BLK
    rows = lambda i: (i, 0)
    return pl.pallas_call(
        _tc_final_body,
        grid=(nblk,),
        in_specs=[pl.BlockSpec((BLK, D), rows),
                  pl.BlockSpec((BLK, D), rows),
                  pl.BlockSpec((BLK, D), rows),
                  pl.BlockSpec((BLK, 1), rows)],
        out_specs=pl.BlockSpec((BLK, D), rows),
        out_shape=jax.ShapeDtypeStruct((N, D), jnp.float32),
    )(base, q0, q1, dis)


def kernel(x, edge_index, edge_attr, W1, b1, W2, b2, Wr1, br1, Wr2, br2):
    npad_e = EPADTOT - E
    row = jnp.concatenate(
        [edge_index[0], jnp.arange(npad_e, dtype=jnp.int32) % N])
    col = jnp.concatenate(
        [edge_index[1],
         N + (jnp.arange(npad_e, dtype=jnp.int32) % (NPAD - N))])
    ew = jnp.concatenate(
        [edge_attr, jnp.zeros((npad_e,), jnp.float32)])

    p, dis_pad = _sc_layer1(row, col, ew, x)
    dis = dis_pad[:N].reshape(N, 1)

    g, base = _tc_mid(x, p[0, :N], p[1, :N], dis,
                      W1, Wr1, W2, Wr2,
                      b1.reshape(1, -1), br1.reshape(1, -1),
                      b2.reshape(1, -1), br2.reshape(1, -1))

    q = _sc_layer2(row, col, ew, g)

    return _tc_final(base, q[0, :N], q[1, :N], dis)


# speedup vs baseline: 2.4871x; 1.0108x over previous
import functools

import jax
import jax.numpy as jnp
from jax import lax
from jax.experimental import pallas as pl
from jax.experimental.pallas import tpu as pltpu
from jax.experimental.pallas import tpu_sc as plsc

N = 10000
E = 320000
D = 128
NPAD = 10240
NC, NS = 2, 16
NW = NC * NS
PADROWS_PER_SUB = NPAD // NS

CH = 64
NBUF = 3
EW_PER_W = 10240
EPADTOT = NW * EW_PER_W
EBLK = 2048
CPB = EBLK // CH
NEBLK = EW_PER_W // EBLK

ED_PER_S = EPADTOT // NS
NDBLK = ED_PER_S // EBLK


def _zero_vec16(ref, nwords):
    z = jnp.zeros((16,), jnp.float32)

    def body(i, _):
        ref[pl.ds(i * 16, 16)] = z
        return 0

    lax.fori_loop(0, nwords // 16, body, 0)


def _zero_rows(ref, nrows):
    z = jnp.zeros((16,), jnp.float32)

    def body(r, _):
        for dblk in range(8):
            ref[r, pl.ds(dblk * 16, 16)] = z
        return 0

    lax.fori_loop(0, nrows, body, 0)


def _rsqrt16(d):
    m = d
    y = jnp.full((16,), 1.0, jnp.float32)
    for _ in range(10):
        c = m > 4.0
        m = jnp.where(c, m * 0.25, m)
        y = jnp.where(c, y * 0.5, y)
    r = jnp.full((16,), 0.7, jnp.float32)
    for _ in range(5):
        r = r * (1.5 - 0.5 * m * r * r)
    return y * r


def _scale_chunk(xb, wbuf, rbuf, ebase, use_dis, dis_vmem):
    def group_body(g, _):
        ev = wbuf[pl.ds(ebase + g * 16, 16)]
        if use_dis:
            rv = rbuf[pl.ds(ebase + g * 16, 16)]
            ev = ev * plsc.load_gather(dis_vmem, [rv])
        for j in range(16):
            er = g * 16 + j
            s = ev[j]
            for dblk in range(8):
                xb[er, pl.ds(dblk * 16, 16)] = xb[er, pl.ds(dblk * 16, 16)] * s
        return 0

    lax.fori_loop(0, CH // 16, group_body, 0)


def _agg_block(tab_hbm, sh_agg, rbuf, wbuf, cix2d, xbs, gsems, ssems,
               use_dis, dis_vmem):
    def gidx(i):
        return rbuf.at[pl.ds(i * CH, CH)]

    for b in range(2):
        pltpu.async_copy(tab_hbm.at[gidx(b)], xbs[b], gsems[b])

    def step(si, _):
        for k in range(NBUF):
            i = si * NBUF + k

            @pl.when(i < CPB)
            def _():
                b = k
                pltpu.make_async_copy(
                    tab_hbm.at[gidx(i)], xbs[b], gsems[b]).wait()
                _scale_chunk(xbs[b], wbuf, rbuf, i * CH, use_dis, dis_vmem)
                pltpu.async_copy(xbs[b], sh_agg.at[cix2d.at[i]], ssems[b],
                                 add=True)

                @pl.when(i + 2 < CPB)
                def _():
                    b2 = (k + 2) % NBUF

                    @pl.when(i >= 1)
                    def _():
                        pltpu.make_async_copy(
                            xbs[b2], sh_agg.at[cix2d.at[i - 1]],
                            ssems[b2]).wait()

                    pltpu.async_copy(
                        tab_hbm.at[gidx(i + 2)], xbs[b2], gsems[b2])
        return 0

    nsteps = (CPB + NBUF - 1) // NBUF
    lax.fori_loop(0, nsteps, step, 0)
    for i in range(CPB - 3, CPB):
        pltpu.make_async_copy(
            xbs[i % NBUF], sh_agg.at[cix2d.at[i]], ssems[i % NBUF]).wait()


def _load_block(row_hbm, col2d_hbm, ew_hbm, rbuf, wbuf, cix2d, off, coff):
    pltpu.sync_copy(col2d_hbm.at[pl.ds(coff, CPB), :], cix2d)
    pltpu.sync_copy(row_hbm.at[pl.ds(off, EBLK)], rbuf)
    pltpu.sync_copy(ew_hbm.at[pl.ds(off, EBLK)], wbuf)


def _agg_all_blocks(tab_hbm, sh_agg, row_hbm, col_hbm, ew_hbm, ebase0,
                    cbase0, rbuf, wbuf, cix2d, xbs, gsems, ssems,
                    use_dis, dis_vmem):
    def blk_body(blk, _):
        _load_block(row_hbm, col_hbm, ew_hbm, rbuf, wbuf, cix2d,
                    ebase0 + blk * EBLK, cbase0 + blk * CPB)
        _agg_block(tab_hbm, sh_agg, rbuf, wbuf, cix2d, xbs, gsems, ssems,
                   use_dis, dis_vmem)
        return 0

    lax.fori_loop(0, NEBLK, blk_body, 0)


def _zero_accum(sh_agg, sh_deg, xb0, wbuf, sid, zero_deg):
    _zero_rows(xb0, CH)
    for b in range(PADROWS_PER_SUB // CH):
        pltpu.sync_copy(
            xb0, sh_agg.at[pl.ds(sid * PADROWS_PER_SUB + b * CH, CH), :])
    if zero_deg:
        _zero_vec16(wbuf, PADROWS_PER_SUB)
        pltpu.sync_copy(wbuf.at[pl.ds(0, PADROWS_PER_SUB)],
                        sh_deg.at[pl.ds(sid * PADROWS_PER_SUB,
                                        PADROWS_PER_SUB)])


def _writeout_partial(sh_agg, out_hbm, cid, sid):
    base = sid * PADROWS_PER_SUB
    pltpu.sync_copy(sh_agg.at[pl.ds(base, PADROWS_PER_SUB), :],
                    out_hbm.at[cid, pl.ds(base, PADROWS_PER_SUB), :])


def _sc_layer1(row, col, ew, x):
    mesh = plsc.VectorSubcoreMesh(core_axis_name="c", subcore_axis_name="s")

    @functools.partial(
        pl.kernel,
        out_type=[jax.ShapeDtypeStruct((NC, NPAD, D), jnp.float32),
                  jax.ShapeDtypeStruct((NPAD,), jnp.float32)],
        mesh=mesh,
        compiler_params=pltpu.CompilerParams(needs_layout_passes=False),
        scratch_types=[
            pltpu.VMEM_SHARED((NPAD, D), jnp.float32),
            pltpu.VMEM_SHARED((NPAD,), jnp.float32),
            pltpu.VMEM((NPAD,), jnp.float32),
            pltpu.VMEM((EBLK,), jnp.int32),
            pltpu.VMEM((EBLK,), jnp.float32),
            pltpu.VMEM((CPB, CH), jnp.int32),
            pltpu.VMEM((CH, D), jnp.float32),
            pltpu.VMEM((CH, D), jnp.float32),
            pltpu.VMEM((CH, D), jnp.float32),
            pltpu.SemaphoreType.DMA,
            pltpu.SemaphoreType.DMA,
            pltpu.SemaphoreType.DMA,
            pltpu.SemaphoreType.DMA,
            pltpu.SemaphoreType.DMA,
            pltpu.SemaphoreType.DMA,
            pltpu.SemaphoreType.DMA,
        ],
    )
    def kern(row_hbm, col_hbm, ew_hbm, x_hbm, p_hbm, dis_hbm,
             sh_agg, sh_deg, dis_vmem, rbuf, wbuf, cix2d, xb0, xb1, xb2,
             g0, g1, g2, s0, s1, s2, dsem):
        cid = lax.axis_index("c")
        sid = lax.axis_index("s")
        wid = cid * NS + sid
        xbs = [xb0, xb1, xb2]
        gsems = [g0, g1, g2]
        ssems = [s0, s1, s2]

        _zero_accum(sh_agg, sh_deg, xb0, wbuf, sid, True)
        plsc.subcore_barrier()

        def deg_block(blk, _):
            off = sid * ED_PER_S + blk * EBLK
            coff = sid * (ED_PER_S // CH) + blk * CPB
            pltpu.sync_copy(col_hbm.at[pl.ds(coff, CPB), :], cix2d)
            pltpu.sync_copy(ew_hbm.at[pl.ds(off, EBLK)], wbuf)
            for k in range(CPB):
                pltpu.async_copy(wbuf.at[pl.ds(k * CH, CH)],
                                 sh_deg.at[cix2d.at[k]], dsem, add=True)
            for k in range(CPB):
                pltpu.make_async_copy(wbuf.at[pl.ds(k * CH, CH)],
                                      sh_deg.at[cix2d.at[k]], dsem).wait()
            return 0

        lax.fori_loop(0, NDBLK, deg_block, 0)
        plsc.subcore_barrier()

        nbase = sid * PADROWS_PER_SUB
        pltpu.sync_copy(sh_deg.at[pl.ds(nbase, PADROWS_PER_SUB)],
                        wbuf.at[pl.ds(0, PADROWS_PER_SUB)])

        def dis_body(i, _):
            dv = wbuf[pl.ds(i * 16, 16)] + 1.0
            wbuf[pl.ds(i * 16, 16)] = _rsqrt16(dv)
            return 0

        lax.fori_loop(0, PADROWS_PER_SUB // 16, dis_body, 0)
        pltpu.sync_copy(wbuf.at[pl.ds(0, PADROWS_PER_SUB)],
                        sh_deg.at[pl.ds(nbase, PADROWS_PER_SUB)])

        @pl.when(cid == 0)
        def _():
            pltpu.sync_copy(wbuf.at[pl.ds(0, PADROWS_PER_SUB)],
                            dis_hbm.at[pl.ds(nbase, PADROWS_PER_SUB)])

        plsc.subcore_barrier()

        pltpu.sync_copy(sh_deg, dis_vmem)
        _agg_all_blocks(x_hbm, sh_agg, row_hbm, col_hbm, ew_hbm,
                        wid * EW_PER_W, wid * (EW_PER_W // CH), rbuf,
                        wbuf, cix2d, xbs, gsems, ssems, True, dis_vmem)
        plsc.subcore_barrier()

        _writeout_partial(sh_agg, p_hbm, cid, sid)

    return kern(row, col, ew, x)


def _sc_layer2(row, col, ew, g):
    mesh = plsc.VectorSubcoreMesh(core_axis_name="c", subcore_axis_name="s")

    @functools.partial(
        pl.kernel,
        out_type=[jax.ShapeDtypeStruct((NC, NPAD, D), jnp.float32)],
        mesh=mesh,
        compiler_params=pltpu.CompilerParams(needs_layout_passes=False),
        scratch_types=[
            pltpu.VMEM_SHARED((NPAD, D), jnp.float32),
            pltpu.VMEM((EBLK,), jnp.int32),
            pltpu.VMEM((EBLK,), jnp.float32),
            pltpu.VMEM((CPB, CH), jnp.int32),
            pltpu.VMEM((CH, D), jnp.float32),
            pltpu.VMEM((CH, D), jnp.float32),
            pltpu.VMEM((CH, D), jnp.float32),
            pltpu.SemaphoreType.DMA,
            pltpu.SemaphoreType.DMA,
            pltpu.SemaphoreType.DMA,
            pltpu.SemaphoreType.DMA,
            pltpu.SemaphoreType.DMA,
            pltpu.SemaphoreType.DMA,
        ],
    )
    def kern(row_hbm, col_hbm, ew_hbm, g_hbm, q_hbm,
             sh_agg, rbuf, wbuf, cix2d, xb0, xb1, xb2,
             g0, g1, g2, s0, s1, s2):
        cid = lax.axis_index("c")
        sid = lax.axis_index("s")
        wid = cid * NS + sid
        xbs = [xb0, xb1, xb2]
        gsems = [g0, g1, g2]
        ssems = [s0, s1, s2]

        _zero_accum(sh_agg, None, xb0, wbuf, sid, False)
        plsc.subcore_barrier()

        _agg_all_blocks(g_hbm, sh_agg, row_hbm, col_hbm, ew_hbm,
                        wid * EW_PER_W, wid * (EW_PER_W // CH), rbuf,
                        wbuf, cix2d, xbs, gsems, ssems, False, None)
        plsc.subcore_barrier()

        _writeout_partial(sh_agg, q_hbm, cid, sid)

    return kern(row, col, ew, g)[0]


BLK = 1000


def _tc_mid_body(x, p0, p1, dis, W1, Wr1, W2, Wr2, b1, br1, b2, br2,
                 g_o, base_o):
    xv = x[...]
    disv = dis[...]
    a = disv * (p0[...] + p1[...]) + (disv * disv) * xv
    h = jnp.maximum(
        jnp.dot(a, W1[...], preferred_element_type=jnp.float32)
        + jnp.dot(xv, Wr1[...], preferred_element_type=jnp.float32)
        + b1[...] + br1[...], 0.0)
    g = disv * jnp.dot(h, W2[...], preferred_element_type=jnp.float32)
    base_o[...] = (jnp.dot(h, Wr2[...], preferred_element_type=jnp.float32)
                   + br2[...] + b2[...] + disv * g)
    g_o[...] = g


def _tc_mid(x, p0, p1, dis, W1, Wr1, W2, Wr2, b1, br1, b2, br2):
    nblk = N // BLK
    rows = lambda i: (i, 0)
    whole = lambda i: (0, 0)
    return pl.pallas_call(
        _tc_mid_body,
        grid=(nblk,),
        in_specs=[
            pl.BlockSpec((BLK, D), rows),
            pl.BlockSpec((BLK, D), rows),
            pl.BlockSpec((BLK, D), rows),
            pl.BlockSpec((BLK, 1), rows),
            pl.BlockSpec((D, 1024), whole),
            pl.BlockSpec((D, 1024), whole),
            pl.BlockSpec((1024, D), whole),
            pl.BlockSpec((1024, D), whole),
            pl.BlockSpec((1, 1024), whole),
            pl.BlockSpec((1, 1024), whole),
            pl.BlockSpec((1, D), whole),
            pl.BlockSpec((1, D), whole),
        ],
        out_specs=[pl.BlockSpec((BLK, D), rows),
                   pl.BlockSpec((BLK, D), rows)],
        out_shape=[jax.ShapeDtypeStruct((N, D), jnp.float32),
                   jax.ShapeDtypeStruct((N, D), jnp.float32)],
    )(x, p0, p1, dis, W1, Wr1, W2, Wr2, b1, br1, b2, br2)


def _tc_final_body(base, q0, q1, dis, out_o):
    out_o[...] = base[...] + dis[...] * (q0[...] + q1[...])


def _tc_final(base, q0, q1, dis):
    nblk = N // BLK
    rows = lambda i: (i, 0)
    return pl.pallas_call(
        _tc_final_body,
        grid=(nblk,),
        in_specs=[pl.BlockSpec((BLK, D), rows),
                  pl.BlockSpec((BLK, D), rows),
                  pl.BlockSpec((BLK, D), rows),
                  pl.BlockSpec((BLK, 1), rows)],
        out_specs=pl.BlockSpec((BLK, D), rows),
        out_shape=jax.ShapeDtypeStruct((N, D), jnp.float32),
    )(base, q0, q1, dis)


def kernel(x, edge_index, edge_attr, W1, b1, W2, b2, Wr1, br1, Wr2, br2):
    npad_e = EPADTOT - E
    row = jnp.concatenate(
        [edge_index[0], jnp.arange(npad_e, dtype=jnp.int32) % N])
    col = jnp.concatenate(
        [edge_index[1],
         N + (jnp.arange(npad_e, dtype=jnp.int32) % (NPAD - N))])
    ew = jnp.concatenate(
        [edge_attr, jnp.zeros((npad_e,), jnp.float32)])
    col = col.reshape(EPADTOT // CH, CH)

    p, dis_pad = _sc_layer1(row, col, ew, x)
    dis = dis_pad[:N].reshape(N, 1)

    g, base = _tc_mid(x, p[0, :N], p[1, :N], dis,
                      W1, Wr1, W2, Wr2,
                      b1.reshape(1, -1), br1.reshape(1, -1),
                      b2.reshape(1, -1), br2.reshape(1, -1))

    q = _sc_layer2(row, col, ew, g)

    return _tc_final(base, q[0, :N], q[1, :N], dis)
